# baseline (speedup 1.0000x reference)
import functools

import jax
import jax.numpy as jnp
import numpy as np
from jax import lax
from jax.experimental import pallas as pl
from jax.experimental.pallas import tpu as pltpu
from jax.experimental.pallas import tpu_sc as plsc

B = 64
D = 768
DM = 2 * D
F = 32768
K = 32
A_END = int(F * 0.05)
B_END = 2 * int(F * 0.05)

NWORKERS = 32
ROWS_PER_W = B // NWORKERS
NCHUNK = 128
CHUNK = F // NCHUNK
NLANEVEC = CHUNK // 16
NGROUPVEC = NCHUNK // 16

FT = 4096


def _enc_body(x_ref, w_ref, b_ref, o_ref):
    acc = jnp.dot(x_ref[:, 0, :], w_ref[0],
                  preferred_element_type=jnp.float32)
    acc += jnp.dot(x_ref[:, 1, :], w_ref[1],
                   preferred_element_type=jnp.float32)
    o_ref[...] = jnp.maximum(acc + b_ref[...], 0.0)


def _encode(x, w_enc, b2):
    return pl.pallas_call(
        _enc_body,
        grid=(F // FT,),
        in_specs=[
            pl.BlockSpec((B, 2, D), lambda i: (0, 0, 0)),
            pl.BlockSpec((2, D, FT), lambda i: (0, 0, i)),
            pl.BlockSpec((1, FT), lambda i: (0, i)),
        ],
        out_specs=pl.BlockSpec((B, FT), lambda i: (0, i)),
        out_shape=jax.ShapeDtypeStruct((B, F), jnp.float32),
        compiler_params=pltpu.CompilerParams(
            dimension_semantics=("arbitrary",)),
    )(x, w_enc, b2)


def _iota16():
    return lax.iota(jnp.int32, 16)


def _splat_i(s):
    return jnp.broadcast_to(s, (16,)).astype(jnp.int32)


def _sstore(ref, i, val):
    mask0 = _iota16() == 0
    plsc.store_scatter(ref, [_splat_i(i)], jnp.broadcast_to(val, (16,)),
                       mask=mask0)


def _sc_body(pre_hbm, wdec_hbm, bdec_hbm, feat_hbm, recon_hbm,
             rowbuf, fbuf, lmax, gmax, wrows, acc, bdecbuf,
             idxbuf, valbuf, sc0buf, sc1buf, gsem, fsem):
    lane = _iota16()
    wid = lax.axis_index("s") * 2 + lax.axis_index("c")

    pltpu.sync_copy(bdec_hbm, bdecbuf)

    def _zb(t, _):
        z = jnp.zeros((16,), jnp.float32)
        for u in range(8):
            fbuf[pl.ds(t * 128 + u * 16, 16)] = z
        return 0
    lax.fori_loop(0, F // 128, _zb, 0)

    def _row(row, _):
        r = wid * ROWS_PER_W + row
        pltpu.sync_copy(pre_hbm.at[r], rowbuf)

        def _p1(g, _):
            base = g * CHUNK
            acc_v = rowbuf[pl.ds(base, 16)]
            for j in range(1, NLANEVEC):
                acc_v = jnp.maximum(acc_v, rowbuf[pl.ds(base + j * 16, 16)])
            lmax[pl.ds(g * 16, 16)] = acc_v
            _sstore(gmax, g, jnp.max(acc_v))
            return 0
        lax.fori_loop(0, NCHUNK, _p1, 0)

        def _ext(k, _):
            gv = [gmax[pl.ds(h * 16, 16)] for h in range(NGROUPVEC)]
            mv = gv[0]
            for h in range(1, NGROUPVEC):
                mv = jnp.maximum(mv, gv[h])
            m = jnp.max(mv)
            cand = jnp.full((16,), 2 * F, jnp.int32)
            for h in range(NGROUPVEC):
                fv = plsc.all_reduce_ffs(gv[h] == m)
                cand = jnp.minimum(
                    cand, jnp.where(fv < 16, h * 16 + fv, 2 * F))
            c = jnp.min(cand)
            lv = lmax[pl.ds(c * 16, 16)]
            l = jnp.max(plsc.all_reduce_ffs(lv == m))
            gidx = c * CHUNK + l + lane * 16
            col = plsc.load_gather(rowbuf, [gidx])
            j = jnp.max(plsc.all_reduce_ffs(col == m))
            e = c * CHUNK + j * 16 + l
            _sstore(idxbuf, k, e)
            _sstore(valbuf, k, m)
            s0 = jnp.where(jnp.logical_and(e >= A_END, e < B_END),
                           jnp.float32(0.0), m)
            s1 = jnp.where(e < A_END, jnp.float32(0.0), m)
            plsc.store_scatter(sc0buf, [k * 16 + lane],
                               jnp.broadcast_to(s0, (16,)))
            plsc.store_scatter(sc1buf, [k * 16 + lane],
                               jnp.broadcast_to(s1, (16,)))
            _sstore(rowbuf, e, jnp.float32(-1.0))
            col2 = jnp.where(lane == j, jnp.float32(-1.0), col)
            nl = jnp.max(col2)
            _sstore(lmax, c * 16 + l, nl)
            lv2 = jnp.where(lane == l, nl, lv)
            _sstore(gmax, c, jnp.max(lv2))
            return 0
        lax.fori_loop(0, K, _ext, 0)

        pltpu.sync_copy(wdec_hbm.at[idxbuf], wrows)

        iv0 = idxbuf[pl.ds(0, 16)]
        iv1 = idxbuf[pl.ds(16, 16)]
        vv0 = valbuf[pl.ds(0, 16)]
        vv1 = valbuf[pl.ds(16, 16)]
        plsc.store_scatter(fbuf, [iv0], vv0)
        plsc.store_scatter(fbuf, [iv1], vv1)
        fcopy = pltpu.async_copy(fbuf, feat_hbm.at[r], fsem)

        for kb in range(K // 8):
            spl0 = [sc0buf[pl.ds((kb * 8 + i) * 16, 16)] for i in range(8)]
            spl1 = [sc1buf[pl.ds((kb * 8 + i) * 16, 16)] for i in range(8)]

            def _mk_tb(spl, mm):
                def _tb(t, _):
                    s = pl.ds(t * 16, 16)
                    av = bdecbuf[mm, s] if kb == 0 else acc[mm, s]
                    for i in range(8):
                        av = av + spl[i] * wrows[kb * 8 + i, mm, s]
                    acc[mm, s] = av
                    return 0
                return _tb
            lax.fori_loop(0, D // 16, _mk_tb(spl0, 0), 0)
            lax.fori_loop(0, D // 16, _mk_tb(spl1, 1), 0)

        pltpu.sync_copy(acc, recon_hbm.at[r])

        fcopy.wait()
        z = jnp.zeros((16,), jnp.float32)
        plsc.store_scatter(fbuf, [iv0], z)
        plsc.store_scatter(fbuf, [iv1], z)
        return 0

    lax.fori_loop(0, ROWS_PER_W, _row, 0)


def _sc_call(pre, wdec2, bdec2):
    mesh = plsc.VectorSubcoreMesh(core_axis_name="c", subcore_axis_name="s",
                                  num_cores=2, num_subcores=16)
    fn = pl.kernel(
        _sc_body,
        out_type=(
            jax.ShapeDtypeStruct((B, F), jnp.float32),
            jax.ShapeDtypeStruct((B, 2, D), jnp.float32),
        ),
        mesh=mesh,
        compiler_params=pltpu.CompilerParams(needs_layout_passes=False),
        scratch_types=[
            pltpu.VMEM((F,), jnp.float32),
            pltpu.VMEM((F,), jnp.float32),
            pltpu.VMEM((F // 16,), jnp.float32),
            pltpu.VMEM((NCHUNK,), jnp.float32),
            pltpu.VMEM((K, 2, D), jnp.float32),
            pltpu.VMEM((2, D), jnp.float32),
            pltpu.VMEM((2, D), jnp.float32),
            pltpu.VMEM((K,), jnp.int32),
            pltpu.VMEM((K,), jnp.float32),
            pltpu.VMEM((K * 16,), jnp.float32),
            pltpu.VMEM((K * 16,), jnp.float32),
            pltpu.SemaphoreType.DMA,
            pltpu.SemaphoreType.DMA,
        ],
    )
    return fn(pre, wdec2, bdec2)


def kernel(x, W_enc, b_enc, W_dec, b_dec):
    pre = _encode(x, W_enc, b_enc.reshape(1, F))
    feats, recon = _sc_call(pre, W_dec, b_dec)
    return recon, feats


# speedup vs baseline: 1.0302x; 1.0302x over previous
import functools

import jax
import jax.numpy as jnp
import numpy as np
from jax import lax
from jax.experimental import pallas as pl
from jax.experimental.pallas import tpu as pltpu
from jax.experimental.pallas import tpu_sc as plsc

B = 64
D = 768
DM = 2 * D
F = 32768
K = 32
A_END = int(F * 0.05)
B_END = 2 * int(F * 0.05)

NWORKERS = 32
ROWS_PER_W = B // NWORKERS
NCHUNK = 128
CHUNK = F // NCHUNK
NLANEVEC = CHUNK // 16
NGROUPVEC = NCHUNK // 16

FT = 2048


def _enc_body(x_ref, w_ref, b_ref, o_ref):
    acc = jnp.dot(x_ref[:, 0, :], w_ref[0],
                  preferred_element_type=jnp.float32)
    acc += jnp.dot(x_ref[:, 1, :], w_ref[1],
                   preferred_element_type=jnp.float32)
    o_ref[...] = jnp.maximum(acc + b_ref[...], 0.0)


def _encode(x, w_enc, b2):
    return pl.pallas_call(
        _enc_body,
        grid=(F // FT,),
        in_specs=[
            pl.BlockSpec((B, 2, D), lambda i: (0, 0, 0)),
            pl.BlockSpec((2, D, FT), lambda i: (0, 0, i)),
            pl.BlockSpec((1, FT), lambda i: (0, i)),
        ],
        out_specs=pl.BlockSpec((B, FT), lambda i: (0, i)),
        out_shape=jax.ShapeDtypeStruct((B, F), jnp.float32),
        compiler_params=pltpu.CompilerParams(
            dimension_semantics=("arbitrary",)),
    )(x, w_enc, b2)


def _iota16():
    return lax.iota(jnp.int32, 16)


def _splat_i(s):
    return jnp.broadcast_to(s, (16,)).astype(jnp.int32)


def _sstore(ref, i, val):
    mask0 = _iota16() == 0
    plsc.store_scatter(ref, [_splat_i(i)], jnp.broadcast_to(val, (16,)),
                       mask=mask0)


def _sc_body(pre_hbm, wdec_hbm, bdec_hbm, feat_hbm, recon_hbm,
             rowbuf, fbuf, lmax, gmax, wrows, acc, bdecbuf,
             idxbuf, valbuf, sc0buf, sc1buf, gsem, fsem):
    lane = _iota16()
    wid = lax.axis_index("s") * 2 + lax.axis_index("c")

    pltpu.sync_copy(bdec_hbm, bdecbuf)

    def _zb(t, _):
        z = jnp.zeros((16,), jnp.float32)
        for u in range(8):
            fbuf[pl.ds(t * 128 + u * 16, 16)] = z
        return 0
    lax.fori_loop(0, F // 128, _zb, 0)

    def _row(row, _):
        r = wid * ROWS_PER_W + row
        pltpu.sync_copy(pre_hbm.at[r], rowbuf)

        def _p1(g, _):
            base = g * CHUNK
            acc_v = rowbuf[pl.ds(base, 16)]
            for j in range(1, NLANEVEC):
                acc_v = jnp.maximum(acc_v, rowbuf[pl.ds(base + j * 16, 16)])
            plsc.store_scatter(lmax, [lane * NCHUNK + g], acc_v)
            return 0
        lax.fori_loop(0, NCHUNK, _p1, 0)

        def _gm(gb, _):
            mv = lmax[pl.ds(gb * 16, 16)]
            for l in range(1, 16):
                mv = jnp.maximum(mv, lmax[pl.ds(l * NCHUNK + gb * 16, 16)])
            gmax[pl.ds(gb * 16, 16)] = mv
            return 0
        lax.fori_loop(0, NCHUNK // 16, _gm, 0)

        def _ext(k, _):
            gv = [gmax[pl.ds(h * 16, 16)] for h in range(NGROUPVEC)]
            mv = gv[0]
            for h in range(1, NGROUPVEC):
                mv = jnp.maximum(mv, gv[h])
            m = jnp.max(mv)
            cand = jnp.full((16,), 2 * F, jnp.int32)
            for h in range(NGROUPVEC):
                fv = plsc.all_reduce_ffs(gv[h] == m)
                cand = jnp.minimum(
                    cand, jnp.where(fv < 16, h * 16 + fv, 2 * F))
            c = jnp.min(cand)
            lv = plsc.load_gather(lmax, [lane * NCHUNK + c])
            l = jnp.max(plsc.all_reduce_ffs(lv == m))
            gidx = c * CHUNK + l + lane * 16
            col = plsc.load_gather(rowbuf, [gidx])
            j = jnp.max(plsc.all_reduce_ffs(col == m))
            e = c * CHUNK + j * 16 + l
            _sstore(idxbuf, k, e)
            _sstore(valbuf, k, m)
            s0 = jnp.where(jnp.logical_and(e >= A_END, e < B_END),
                           jnp.float32(0.0), m)
            s1 = jnp.where(e < A_END, jnp.float32(0.0), m)
            plsc.store_scatter(sc0buf, [k * 16 + lane],
                               jnp.broadcast_to(s0, (16,)))
            plsc.store_scatter(sc1buf, [k * 16 + lane],
                               jnp.broadcast_to(s1, (16,)))
            _sstore(rowbuf, e, jnp.float32(-1.0))
            col2 = jnp.where(lane == j, jnp.float32(-1.0), col)
            nl = jnp.max(col2)
            _sstore(lmax, l * NCHUNK + c, nl)
            lv2 = jnp.where(lane == l, nl, lv)
            _sstore(gmax, c, jnp.max(lv2))
            return 0
        lax.fori_loop(0, K, _ext, 0)

        pltpu.sync_copy(wdec_hbm.at[idxbuf], wrows)

        iv0 = idxbuf[pl.ds(0, 16)]
        iv1 = idxbuf[pl.ds(16, 16)]
        vv0 = valbuf[pl.ds(0, 16)]
        vv1 = valbuf[pl.ds(16, 16)]
        plsc.store_scatter(fbuf, [iv0], vv0)
        plsc.store_scatter(fbuf, [iv1], vv1)
        fcopy = pltpu.async_copy(fbuf, feat_hbm.at[r], fsem)

        for kb in range(K // 16):
            spl0 = [sc0buf[pl.ds((kb * 16 + i) * 16, 16)] for i in range(16)]
            spl1 = [sc1buf[pl.ds((kb * 16 + i) * 16, 16)] for i in range(16)]

            def _mk_tb(spl, mm):
                def _tb(t, _):
                    s = pl.ds(t * 16, 16)
                    av = bdecbuf[mm, s] if kb == 0 else acc[mm, s]
                    for i in range(16):
                        av = av + spl[i] * wrows[kb * 16 + i, mm, s]
                    acc[mm, s] = av
                    return 0
                return _tb
            lax.fori_loop(0, D // 16, _mk_tb(spl0, 0), 0)
            lax.fori_loop(0, D // 16, _mk_tb(spl1, 1), 0)

        pltpu.sync_copy(acc, recon_hbm.at[r])

        fcopy.wait()
        z = jnp.zeros((16,), jnp.float32)
        plsc.store_scatter(fbuf, [iv0], z)
        plsc.store_scatter(fbuf, [iv1], z)
        return 0

    lax.fori_loop(0, ROWS_PER_W, _row, 0)


def _sc_call(pre, wdec2, bdec2):
    mesh = plsc.VectorSubcoreMesh(core_axis_name="c", subcore_axis_name="s",
                                  num_cores=2, num_subcores=16)
    fn = pl.kernel(
        _sc_body,
        out_type=(
            jax.ShapeDtypeStruct((B, F), jnp.float32),
            jax.ShapeDtypeStruct((B, 2, D), jnp.float32),
        ),
        mesh=mesh,
        compiler_params=pltpu.CompilerParams(needs_layout_passes=False),
        scratch_types=[
            pltpu.VMEM((F,), jnp.float32),
            pltpu.VMEM((F,), jnp.float32),
            pltpu.VMEM((F // 16,), jnp.float32),
            pltpu.VMEM((NCHUNK,), jnp.float32),
            pltpu.VMEM((K, 2, D), jnp.float32),
            pltpu.VMEM((2, D), jnp.float32),
            pltpu.VMEM((2, D), jnp.float32),
            pltpu.VMEM((K,), jnp.int32),
            pltpu.VMEM((K,), jnp.float32),
            pltpu.VMEM((K * 16,), jnp.float32),
            pltpu.VMEM((K * 16,), jnp.float32),
            pltpu.SemaphoreType.DMA,
            pltpu.SemaphoreType.DMA,
        ],
    )
    return fn(pre, wdec2, bdec2)


def kernel(x, W_enc, b_enc, W_dec, b_dec):
    pre = _encode(x, W_enc, b_enc.reshape(1, F))
    feats, recon = _sc_call(pre, W_dec, b_dec)
    return recon, feats


# speedup vs baseline: 1.0505x; 1.0197x over previous
import functools

import jax
import jax.numpy as jnp
import numpy as np
from jax import lax
from jax.experimental import pallas as pl
from jax.experimental.pallas import tpu as pltpu
from jax.experimental.pallas import tpu_sc as plsc

B = 64
D = 768
DM = 2 * D
F = 32768
K = 32
A_END = int(F * 0.05)
B_END = 2 * int(F * 0.05)

NWORKERS = 32
ROWS_PER_W = B // NWORKERS
NCHUNK = 128
CHUNK = F // NCHUNK
NLANEVEC = CHUNK // 16
NGROUPVEC = NCHUNK // 16

FT = 2048


def _enc_body(x_ref, w_ref, b_ref, o_ref):
    acc = jnp.dot(x_ref[:, 0, :], w_ref[0],
                  preferred_element_type=jnp.float32)
    acc += jnp.dot(x_ref[:, 1, :], w_ref[1],
                   preferred_element_type=jnp.float32)
    o_ref[...] = jnp.maximum(acc + b_ref[...], 0.0)


def _encode(x, w_enc, b2):
    return pl.pallas_call(
        _enc_body,
        grid=(F // FT,),
        in_specs=[
            pl.BlockSpec((B, 2, D), lambda i: (0, 0, 0)),
            pl.BlockSpec((2, D, FT), lambda i: (0, 0, i)),
            pl.BlockSpec((1, FT), lambda i: (0, i)),
        ],
        out_specs=pl.BlockSpec((B, FT), lambda i: (0, i)),
        out_shape=jax.ShapeDtypeStruct((B, F), jnp.float32),
        compiler_params=pltpu.CompilerParams(
            dimension_semantics=("arbitrary",)),
    )(x, w_enc, b2)


def _iota16():
    return lax.iota(jnp.int32, 16)


def _splat_i(s):
    return jnp.broadcast_to(s, (16,)).astype(jnp.int32)


def _sstore(ref, i, val):
    mask0 = _iota16() == 0
    plsc.store_scatter(ref, [_splat_i(i)], jnp.broadcast_to(val, (16,)),
                       mask=mask0)


def _bfly_max(v):
    idx = _iota16()
    dnums = lax.GatherDimensionNumbers(
        offset_dims=(), collapsed_slice_dims=(0,), start_index_map=(0,))
    for sh in (8, 4, 2, 1):
        perm = lax.gather(v, (idx ^ sh)[:, None], dnums, (1,),
                          mode=lax.GatherScatterMode.PROMISE_IN_BOUNDS)
        v = jnp.maximum(v, perm)
    return v


def _sc_body(pre_hbm, wdec_hbm, bdec_hbm, feat_hbm, recon_hbm,
             rowbuf, fbuf, lmax, gmax, wrows, acc, bdecbuf,
             idxbuf, valbuf, sc0buf, sc1buf, gsem, fsem):
    lane = _iota16()
    wid = lax.axis_index("s") * 2 + lax.axis_index("c")

    pltpu.sync_copy(bdec_hbm, bdecbuf)

    def _zb(t, _):
        z = jnp.zeros((16,), jnp.float32)
        for u in range(8):
            fbuf[pl.ds(t * 128 + u * 16, 16)] = z
        return 0
    lax.fori_loop(0, F // 128, _zb, 0)

    def _row(row, _):
        r = wid * ROWS_PER_W + row
        pltpu.sync_copy(pre_hbm.at[r], rowbuf)

        def _p1(g, _):
            base = g * CHUNK
            acc_v = rowbuf[pl.ds(base, 16)]
            for j in range(1, NLANEVEC):
                acc_v = jnp.maximum(acc_v, rowbuf[pl.ds(base + j * 16, 16)])
            plsc.store_scatter(lmax, [lane * NCHUNK + g], acc_v)
            return 0
        lax.fori_loop(0, NCHUNK, _p1, 0)

        def _gm(gb, _):
            mv = lmax[pl.ds(gb * 16, 16)]
            for l in range(1, 16):
                mv = jnp.maximum(mv, lmax[pl.ds(l * NCHUNK + gb * 16, 16)])
            gmax[pl.ds(gb * 16, 16)] = mv
            return 0
        lax.fori_loop(0, NCHUNK // 16, _gm, 0)

        def _ext(k, _):
            gv = [gmax[pl.ds(h * 16, 16)] for h in range(NGROUPVEC)]
            mv = gv[0]
            for h in range(1, NGROUPVEC):
                mv = jnp.maximum(mv, gv[h])
            m = _bfly_max(mv)
            cand = jnp.full((16,), 2 * F, jnp.int32)
            for h in range(NGROUPVEC):
                fv = plsc.all_reduce_ffs(gv[h] == m)
                cand = jnp.minimum(
                    cand, jnp.where(fv < 16, h * 16 + fv, 2 * F))
            c = cand
            lv = plsc.load_gather(lmax, [lane * NCHUNK + c])
            l = plsc.all_reduce_ffs(lv == m)
            gidx = c * CHUNK + l + lane * 16
            col = plsc.load_gather(rowbuf, [gidx])
            j = plsc.all_reduce_ffs(col == m)
            e = c * CHUNK + j * 16 + l
            _sstore(idxbuf, k, e)
            _sstore(valbuf, k, m)
            s0 = jnp.where(jnp.logical_and(e >= A_END, e < B_END),
                           jnp.float32(0.0), m)
            s1 = jnp.where(e < A_END, jnp.float32(0.0), m)
            plsc.store_scatter(sc0buf, [k * 16 + lane],
                               jnp.broadcast_to(s0, (16,)))
            plsc.store_scatter(sc1buf, [k * 16 + lane],
                               jnp.broadcast_to(s1, (16,)))
            _sstore(rowbuf, e, jnp.float32(-1.0))
            col2 = jnp.where(lane == j, jnp.float32(-1.0), col)
            nl = _bfly_max(col2)
            _sstore(lmax, l * NCHUNK + c, nl)
            lv2 = jnp.where(lane == l, nl, lv)
            _sstore(gmax, c, _bfly_max(lv2))
            return 0
        lax.fori_loop(0, K, _ext, 0)

        pltpu.sync_copy(wdec_hbm.at[idxbuf], wrows)

        iv0 = idxbuf[pl.ds(0, 16)]
        iv1 = idxbuf[pl.ds(16, 16)]
        vv0 = valbuf[pl.ds(0, 16)]
        vv1 = valbuf[pl.ds(16, 16)]
        plsc.store_scatter(fbuf, [iv0], vv0)
        plsc.store_scatter(fbuf, [iv1], vv1)
        fcopy = pltpu.async_copy(fbuf, feat_hbm.at[r], fsem)

        for kb in range(K // 16):
            spl0 = [sc0buf[pl.ds((kb * 16 + i) * 16, 16)] for i in range(16)]
            spl1 = [sc1buf[pl.ds((kb * 16 + i) * 16, 16)] for i in range(16)]

            def _mk_tb(spl, mm):
                def _tb(t, _):
                    s = pl.ds(t * 16, 16)
                    av = bdecbuf[mm, s] if kb == 0 else acc[mm, s]
                    for i in range(16):
                        av = av + spl[i] * wrows[kb * 16 + i, mm, s]
                    acc[mm, s] = av
                    return 0
                return _tb
            lax.fori_loop(0, D // 16, _mk_tb(spl0, 0), 0)
            lax.fori_loop(0, D // 16, _mk_tb(spl1, 1), 0)

        pltpu.sync_copy(acc, recon_hbm.at[r])

        fcopy.wait()
        z = jnp.zeros((16,), jnp.float32)
        plsc.store_scatter(fbuf, [iv0], z)
        plsc.store_scatter(fbuf, [iv1], z)
        return 0

    lax.fori_loop(0, ROWS_PER_W, _row, 0)


def _sc_call(pre, wdec2, bdec2):
    mesh = plsc.VectorSubcoreMesh(core_axis_name="c", subcore_axis_name="s",
                                  num_cores=2, num_subcores=16)
    fn = pl.kernel(
        _sc_body,
        out_type=(
            jax.ShapeDtypeStruct((B, F), jnp.float32),
            jax.ShapeDtypeStruct((B, 2, D), jnp.float32),
        ),
        mesh=mesh,
        compiler_params=pltpu.CompilerParams(needs_layout_passes=False),
        scratch_types=[
            pltpu.VMEM((F,), jnp.float32),
            pltpu.VMEM((F,), jnp.float32),
            pltpu.VMEM((F // 16,), jnp.float32),
            pltpu.VMEM((NCHUNK,), jnp.float32),
            pltpu.VMEM((K, 2, D), jnp.float32),
            pltpu.VMEM((2, D), jnp.float32),
            pltpu.VMEM((2, D), jnp.float32),
            pltpu.VMEM((K,), jnp.int32),
            pltpu.VMEM((K,), jnp.float32),
            pltpu.VMEM((K * 16,), jnp.float32),
            pltpu.VMEM((K * 16,), jnp.float32),
            pltpu.SemaphoreType.DMA,
            pltpu.SemaphoreType.DMA,
        ],
    )
    return fn(pre, wdec2, bdec2)


def kernel(x, W_enc, b_enc, W_dec, b_dec):
    pre = _encode(x, W_enc, b_enc.reshape(1, F))
    feats, recon = _sc_call(pre, W_dec, b_dec)
    return recon, feats
